# baseline (speedup 1.0000x reference)
import functools

import jax
import jax.numpy as jnp
from jax import lax
from jax.experimental import pallas as pl
from jax.experimental.pallas import tpu as pltpu
from jax.experimental.pallas import tpu_sc as plsc

N = 10000
D = 128
E = 320000
NC = 2
NS = 16
NW = NC * NS
CHUNK = 192
EPW = E // NW
NCH = -(-EPW // CHUNK)
EPW_PAD = NCH * CHUNK
RPT = 626
R = RPT * NS
JUNK = N
BLK = 1000


def _sc_mesh():
    return plsc.VectorSubcoreMesh(core_axis_name="c", subcore_axis_name="s",
                                  num_cores=NC, num_subcores=NS)


CH1 = 128
NCH1 = -(-EPW // CH1)
EPW_PAD1 = NCH1 * CH1


def _deg_body(dst_hbm, ones_hbm, zeros_hbm, out_hbm, didx, ones_v, acc):
    c = lax.axis_index("c")
    s = lax.axis_index("s")
    w = c * NS + s
    pltpu.sync_copy(zeros_hbm, acc.at[pl.ds(s * RPT, RPT)])
    pltpu.sync_copy(dst_hbm.at[w], didx)
    pltpu.sync_copy(ones_hbm, ones_v)
    plsc.subcore_barrier()

    for j in range(NCH1):
        pltpu.sync_copy(ones_v, acc.at[didx.at[j]], add=True)
    plsc.subcore_barrier()
    pltpu.sync_copy(acc.at[pl.ds(s * RPT, RPT)],
                    out_hbm.at[c, pl.ds(s * RPT, RPT)])


@functools.lru_cache(maxsize=None)
def _deg_call():
    return pl.kernel(
        _deg_body,
        out_type=jax.ShapeDtypeStruct((NC, R, 16), jnp.float32),
        mesh=_sc_mesh(),
        scratch_types=[
            pltpu.VMEM((NCH1, CH1), jnp.int32),
            pltpu.VMEM((CH1, 16), jnp.float32),
            pltpu.VMEM_SHARED((R, 16), jnp.float32),
        ],
        compiler_params=pltpu.CompilerParams(use_tc_tiling_on_sc=False),
    )


def _msg_body(g_hbm, src_hbm, dst_hbm, zeros_hbm, out_hbm,
              sidx0, sidx1, didx0, didx1, gbuf0, gbuf1,
              semg0, semg1, semi0, semi1, semid0, semid1,
              semsc0, semsc1, acc):
    c = lax.axis_index("c")
    s = lax.axis_index("s")
    w = c * NS + s
    pltpu.sync_copy(zeros_hbm, acc.at[pl.ds(s * RPT, RPT)])
    sbufs = (sidx0, sidx1)
    dbufs = (didx0, didx1)
    gbufs = (gbuf0, gbuf1)
    semg = (semg0, semg1)
    semi = (semi0, semi1)
    semid = (semid0, semid1)
    semsc = (semsc0, semsc1)
    pltpu.sync_copy(src_hbm.at[w, 0], sidx0)
    pltpu.sync_copy(dst_hbm.at[w, 0], didx0)
    plsc.subcore_barrier()
    pltpu.async_copy(g_hbm.at[sidx0], gbuf0, semg0)
    if NCH > 1:
        pltpu.async_copy(src_hbm.at[w, 1], sidx1, semi1)
    scat = [None] * NCH
    for j in range(NCH):
        p, q = j % 2, (j + 1) % 2
        if j >= 1:
            scat[j - 1].wait()
        if j + 1 < NCH:
            pltpu.async_copy(dst_hbm.at[w, j + 1], dbufs[q], semid[q])
            pltpu.make_async_copy(src_hbm.at[w, j + 1], sbufs[q],
                                  semi[q]).wait()
            pltpu.async_copy(g_hbm.at[sbufs[q]], gbufs[q], semg[q])
        pltpu.make_async_copy(g_hbm.at[sbufs[p]], gbufs[p], semg[p]).wait()
        if j + 2 < NCH:
            pltpu.async_copy(src_hbm.at[w, j + 2], sbufs[p], semi[p])
        if j >= 1:
            pltpu.make_async_copy(dst_hbm.at[w, j], dbufs[p],
                                  semid[p]).wait()
        scat[j] = pltpu.make_async_copy(gbufs[p], acc.at[dbufs[p]],
                                        semsc[p])
        scat[j].start(add=True)
    scat[NCH - 1].wait()
    plsc.subcore_barrier()
    pltpu.sync_copy(acc.at[pl.ds(s * RPT, RPT)],
                    out_hbm.at[c, pl.ds(s * RPT, RPT)])


@functools.lru_cache(maxsize=None)
def _msg_call():
    return pl.kernel(
        _msg_body,
        out_type=jax.ShapeDtypeStruct((NC, R, D), jnp.float32),
        mesh=_sc_mesh(),
        scratch_types=[
            pltpu.VMEM((CHUNK,), jnp.int32),
            pltpu.VMEM((CHUNK,), jnp.int32),
            pltpu.VMEM((CHUNK,), jnp.int32),
            pltpu.VMEM((CHUNK,), jnp.int32),
            pltpu.VMEM((CHUNK, D), jnp.float32),
            pltpu.VMEM((CHUNK, D), jnp.float32),
            pltpu.SemaphoreType.DMA,
            pltpu.SemaphoreType.DMA,
            pltpu.SemaphoreType.DMA,
            pltpu.SemaphoreType.DMA,
            pltpu.SemaphoreType.DMA,
            pltpu.SemaphoreType.DMA,
            pltpu.SemaphoreType.DMA,
            pltpu.SemaphoreType.DMA,
            pltpu.VMEM_SHARED((R, D), jnp.float32),
        ],
        compiler_params=pltpu.CompilerParams(use_tc_tiling_on_sc=False),
    )


def _dense_body(xp_ref, xn_ref, d0_ref, d1_ref, gam_ref, bet_ref, w_ref,
                g_ref, dinv_ref):
    gam = gam_ref[0, :]
    bet = bet_ref[0, :]

    def ln(v):
        mu = jnp.mean(v, axis=1, keepdims=True)
        vc = v - mu
        var = jnp.mean(vc * vc, axis=1, keepdims=True)
        return gam * vc * lax.rsqrt(var + 1e-5) + bet

    yp = ln(xp_ref[...])
    yn = ln(xn_ref[...])
    h = (jnp.dot(yp, w_ref[:D, :], precision="highest",
                 preferred_element_type=jnp.float32)
         + jnp.dot(yn, w_ref[D:, :], precision="highest",
                   preferred_element_type=jnp.float32))
    deg = d0_ref[0, :, 0:1] + d1_ref[0, :, 0:1] + 1.0
    dinv = lax.rsqrt(deg)
    g_ref[...] = h * dinv
    dinv_ref[...] = dinv


def _dense_call(x_prev, x_next, degp, ln_gamma, ln_beta, W):
    grid = N // BLK
    return pl.pallas_call(
        _dense_body,
        grid=(grid,),
        in_specs=[
            pl.BlockSpec((BLK, D), lambda i: (i, 0)),
            pl.BlockSpec((BLK, D), lambda i: (i, 0)),
            pl.BlockSpec((1, BLK, 16), lambda i: (0, i, 0)),
            pl.BlockSpec((1, BLK, 16), lambda i: (1, i, 0)),
            pl.BlockSpec((1, D), lambda i: (0, 0)),
            pl.BlockSpec((1, D), lambda i: (0, 0)),
            pl.BlockSpec((2 * D, D), lambda i: (0, 0)),
        ],
        out_specs=[
            pl.BlockSpec((BLK, D), lambda i: (i, 0)),
            pl.BlockSpec((BLK, 1), lambda i: (i, 0)),
        ],
        out_shape=[
            jax.ShapeDtypeStruct((N, D), jnp.float32),
            jax.ShapeDtypeStruct((N, 1), jnp.float32),
        ],
    )(x_prev, x_next, degp, degp, ln_gamma.reshape(1, D),
      ln_beta.reshape(1, D), W)


def _comb_body(a0_ref, a1_ref, g_ref, dinv_ref, b_ref, out_ref):
    out_ref[...] = (dinv_ref[...]
                    * (a0_ref[0] + a1_ref[0] + g_ref[...])
                    + b_ref[0, :])


def _comb_call(accp, g, dinv, b):
    grid = N // BLK
    return pl.pallas_call(
        _comb_body,
        grid=(grid,),
        in_specs=[
            pl.BlockSpec((1, BLK, D), lambda i: (0, i, 0)),
            pl.BlockSpec((1, BLK, D), lambda i: (1, i, 0)),
            pl.BlockSpec((BLK, D), lambda i: (i, 0)),
            pl.BlockSpec((BLK, 1), lambda i: (i, 0)),
            pl.BlockSpec((1, D), lambda i: (0, 0)),
        ],
        out_specs=pl.BlockSpec((BLK, D), lambda i: (i, 0)),
        out_shape=jax.ShapeDtypeStruct((N, D), jnp.float32),
    )(accp, accp, g, dinv, b.reshape(1, D))


@jax.jit
def kernel(x_prev, x_same, x_next, edge_index, ln_gamma, ln_beta, W, b):
    del x_same
    src = edge_index[0].reshape(NW, EPW)
    dst = edge_index[1].reshape(NW, EPW)
    pad = EPW_PAD - EPW
    srcp = jnp.pad(src, ((0, 0), (0, pad))).reshape(NW, NCH, CHUNK)
    dstp = jnp.pad(dst, ((0, 0), (0, pad)),
                   constant_values=JUNK).reshape(NW, NCH, CHUNK)
    pad1 = EPW_PAD1 - EPW
    dstp1 = jnp.pad(dst, ((0, 0), (0, pad1)),
                    constant_values=JUNK).reshape(NW, NCH1, CH1)

    ones16 = jnp.ones((CH1, 16), jnp.float32)
    z16 = jnp.zeros((RPT, 16), jnp.float32)
    zD = jnp.zeros((RPT, D), jnp.float32)

    degp = _deg_call()(dstp1, ones16, z16)
    g, dinv = _dense_call(x_prev, x_next, degp, ln_gamma, ln_beta, W)
    accp = _msg_call()(g, srcp, dstp, zD)
    return _comb_call(accp, g, dinv, b)


# speedup vs baseline: 1.2290x; 1.2290x over previous
import functools

import jax
import jax.numpy as jnp
from jax import lax
from jax.experimental import pallas as pl
from jax.experimental.pallas import tpu as pltpu
from jax.experimental.pallas import tpu_sc as plsc

N = 10000
D = 128
E = 320000
NC = 2
NS = 16
NW = NC * NS
CHUNK = 128
EPW = E // NW
NCH = -(-EPW // CHUNK)
EPW_PAD = NCH * CHUNK
RPT = 626
R = RPT * NS
JUNK = N
BLK = 1000


def _sc_mesh():
    return plsc.VectorSubcoreMesh(core_axis_name="c", subcore_axis_name="s",
                                  num_cores=NC, num_subcores=NS)


def _deg_body(dst_hbm, ones_hbm, zeros_hbm, out_hbm, didx, ones_v, acc):
    c = lax.axis_index("c")
    s = lax.axis_index("s")
    w = c * NS + s
    pltpu.sync_copy(zeros_hbm, acc.at[pl.ds(s * RPT, RPT)])
    pltpu.sync_copy(dst_hbm.at[w], didx)
    pltpu.sync_copy(ones_hbm, ones_v)
    plsc.subcore_barrier()

    for j in range(NCH):
        pltpu.sync_copy(ones_v, acc.at[didx.at[j]], add=True)
    plsc.subcore_barrier()
    pltpu.sync_copy(acc.at[pl.ds(s * RPT, RPT)],
                    out_hbm.at[c, pl.ds(s * RPT, RPT)])


@functools.lru_cache(maxsize=None)
def _deg_call():
    return pl.kernel(
        _deg_body,
        out_type=jax.ShapeDtypeStruct((NC, R, 16), jnp.float32),
        mesh=_sc_mesh(),
        scratch_types=[
            pltpu.VMEM((NCH, CHUNK), jnp.int32),
            pltpu.VMEM((CHUNK, 16), jnp.float32),
            pltpu.VMEM_SHARED((R, 16), jnp.float32),
        ],
        compiler_params=pltpu.CompilerParams(use_tc_tiling_on_sc=False),
    )


def _msg_body(g_hbm, src_hbm, dst_hbm, zeros_hbm, out_hbm,
              sidx0, sidx1, didx, gbuf0, gbuf1, semg0, semg1, semi0, semi1,
              semsc0, semsc1, acc):
    c = lax.axis_index("c")
    s = lax.axis_index("s")
    w = c * NS + s
    pltpu.sync_copy(zeros_hbm, acc.at[pl.ds(s * RPT, RPT)])
    pltpu.sync_copy(dst_hbm.at[w], didx)
    sbufs = (sidx0, sidx1)
    gbufs = (gbuf0, gbuf1)
    semg = (semg0, semg1)
    semi = (semi0, semi1)
    pltpu.sync_copy(src_hbm.at[w, 0], sidx0)
    plsc.subcore_barrier()
    pltpu.async_copy(g_hbm.at[sidx0], gbuf0, semg0)
    if NCH > 1:
        pltpu.async_copy(src_hbm.at[w, 1], sidx1, semi1)
    semsc = (semsc0, semsc1)
    scat = [None] * NCH
    for j in range(NCH):
        p, q = j % 2, (j + 1) % 2
        if j >= 1:
            scat[j - 1].wait()
        if j + 1 < NCH:
            pltpu.make_async_copy(src_hbm.at[w, j + 1], sbufs[q],
                                  semi[q]).wait()
            pltpu.async_copy(g_hbm.at[sbufs[q]], gbufs[q], semg[q])
        pltpu.make_async_copy(g_hbm.at[sbufs[p]], gbufs[p], semg[p]).wait()
        scat[j] = pltpu.make_async_copy(gbufs[p], acc.at[didx.at[j]],
                                        semsc[p])
        scat[j].start(add=True)
        if j + 2 < NCH:
            pltpu.async_copy(src_hbm.at[w, j + 2], sbufs[p], semi[p])
    scat[NCH - 1].wait()
    plsc.subcore_barrier()
    pltpu.sync_copy(acc.at[pl.ds(s * RPT, RPT)],
                    out_hbm.at[c, pl.ds(s * RPT, RPT)])


@functools.lru_cache(maxsize=None)
def _msg_call():
    return pl.kernel(
        _msg_body,
        out_type=jax.ShapeDtypeStruct((NC, R, D), jnp.float32),
        mesh=_sc_mesh(),
        scratch_types=[
            pltpu.VMEM((CHUNK,), jnp.int32),
            pltpu.VMEM((CHUNK,), jnp.int32),
            pltpu.VMEM((NCH, CHUNK), jnp.int32),
            pltpu.VMEM((CHUNK, D), jnp.float32),
            pltpu.VMEM((CHUNK, D), jnp.float32),
            pltpu.SemaphoreType.DMA,
            pltpu.SemaphoreType.DMA,
            pltpu.SemaphoreType.DMA,
            pltpu.SemaphoreType.DMA,
            pltpu.SemaphoreType.DMA,
            pltpu.SemaphoreType.DMA,
            pltpu.VMEM_SHARED((R, D), jnp.float32),
        ],
        compiler_params=pltpu.CompilerParams(use_tc_tiling_on_sc=False),
    )


def _dense_body(xp_ref, xn_ref, d0_ref, d1_ref, gam_ref, bet_ref, w_ref,
                g_ref, dinv_ref):
    gam = gam_ref[0, :]
    bet = bet_ref[0, :]

    def ln(v):
        mu = jnp.mean(v, axis=1, keepdims=True)
        vc = v - mu
        var = jnp.mean(vc * vc, axis=1, keepdims=True)
        return gam * vc * lax.rsqrt(var + 1e-5) + bet

    yp = ln(xp_ref[...])
    yn = ln(xn_ref[...])
    h = (jnp.dot(yp, w_ref[:D, :], precision="highest",
                 preferred_element_type=jnp.float32)
         + jnp.dot(yn, w_ref[D:, :], precision="highest",
                   preferred_element_type=jnp.float32))
    deg = d0_ref[0, :, 0:1] + d1_ref[0, :, 0:1] + 1.0
    dinv = lax.rsqrt(deg)
    g_ref[...] = h * dinv
    dinv_ref[...] = dinv


def _dense_call(x_prev, x_next, degp, ln_gamma, ln_beta, W):
    grid = N // BLK
    return pl.pallas_call(
        _dense_body,
        grid=(grid,),
        in_specs=[
            pl.BlockSpec((BLK, D), lambda i: (i, 0)),
            pl.BlockSpec((BLK, D), lambda i: (i, 0)),
            pl.BlockSpec((1, BLK, 16), lambda i: (0, i, 0)),
            pl.BlockSpec((1, BLK, 16), lambda i: (1, i, 0)),
            pl.BlockSpec((1, D), lambda i: (0, 0)),
            pl.BlockSpec((1, D), lambda i: (0, 0)),
            pl.BlockSpec((2 * D, D), lambda i: (0, 0)),
        ],
        out_specs=[
            pl.BlockSpec((BLK, D), lambda i: (i, 0)),
            pl.BlockSpec((BLK, 1), lambda i: (i, 0)),
        ],
        out_shape=[
            jax.ShapeDtypeStruct((N, D), jnp.float32),
            jax.ShapeDtypeStruct((N, 1), jnp.float32),
        ],
    )(x_prev, x_next, degp, degp, ln_gamma.reshape(1, D),
      ln_beta.reshape(1, D), W)


def _comb_body(a0_ref, a1_ref, g_ref, dinv_ref, b_ref, out_ref):
    out_ref[...] = (dinv_ref[...]
                    * (a0_ref[0] + a1_ref[0] + g_ref[...])
                    + b_ref[0, :])


def _comb_call(accp, g, dinv, b):
    grid = N // BLK
    return pl.pallas_call(
        _comb_body,
        grid=(grid,),
        in_specs=[
            pl.BlockSpec((1, BLK, D), lambda i: (0, i, 0)),
            pl.BlockSpec((1, BLK, D), lambda i: (1, i, 0)),
            pl.BlockSpec((BLK, D), lambda i: (i, 0)),
            pl.BlockSpec((BLK, 1), lambda i: (i, 0)),
            pl.BlockSpec((1, D), lambda i: (0, 0)),
        ],
        out_specs=pl.BlockSpec((BLK, D), lambda i: (i, 0)),
        out_shape=jax.ShapeDtypeStruct((N, D), jnp.float32),
    )(accp, accp, g, dinv, b.reshape(1, D))


@jax.jit
def kernel(x_prev, x_same, x_next, edge_index, ln_gamma, ln_beta, W, b):
    del x_same
    src = edge_index[0].reshape(NW, EPW)
    dst = edge_index[1].reshape(NW, EPW)
    pad = EPW_PAD - EPW
    srcp = jnp.pad(src, ((0, 0), (0, pad))).reshape(NW, NCH, CHUNK)
    dstp = jnp.pad(dst, ((0, 0), (0, pad)),
                   constant_values=JUNK).reshape(NW, NCH, CHUNK)

    ones16 = jnp.ones((CHUNK, 16), jnp.float32)
    z16 = jnp.zeros((RPT, 16), jnp.float32)
    zD = jnp.zeros((RPT, D), jnp.float32)

    degp = _deg_call()(dstp, ones16, z16)
    g, dinv = _dense_call(x_prev, x_next, degp, ln_gamma, ln_beta, W)
    accp = _msg_call()(g, srcp, dstp, zD)
    return _comb_call(accp, g, dinv, b)
